# Initial kernel scaffold; baseline (speedup 1.0000x reference)
import jax
import jax.numpy as jnp
from jax.experimental import pallas as pl


def kernel(x, adj, W1l, W1r, att1, b1, bn_g, bn_b, W2l, W2r, att2, b2, W3l, W3r, att3, b3):
    raise NotImplementedError("write your pallas kernel here")


# speedup vs baseline: 5.5345x; 5.5345x over previous
import functools
import jax
import jax.numpy as jnp
from jax import lax
from jax.experimental import pallas as pl
from jax.experimental.pallas import tpu as pltpu
from jax.experimental.pallas import tpu_sc as plsc

_N = 10000
_E = 320000
_ETOT = _E + _N
_NW = 32
_B = 128
_NB = -(-_ETOT // (_NW * _B))
_EPW = _NB * _B
_NPAD = 10240
_SL = _NPAD // 16

_mesh = plsc.VectorSubcoreMesh(core_axis_name="c", subcore_axis_name="s",
                               num_cores=2, num_subcores=16)
_params = pltpu.CompilerParams(needs_layout_passes=False)


def _f32(*shape):
    return jax.ShapeDtypeStruct(shape, jnp.float32)


def _make_kernel_a(H):
    HC = H // 16

    @functools.partial(
        pl.kernel,
        mesh=_mesh,
        compiler_params=_params,
        out_type=[_f32(_NW, _EPW), _f32(2, _NPAD)],
        scratch_types=[
            pltpu.VMEM((_B,), jnp.int32),
            pltpu.VMEM((_B,), jnp.int32),
            pltpu.VMEM((_B, H), jnp.float32),
            pltpu.VMEM((_B, H), jnp.float32),
            pltpu.VMEM((H,), jnp.float32),
            pltpu.VMEM((_EPW,), jnp.float32),
            pltpu.VMEM((_NPAD,), jnp.float32),
            pltpu.VMEM((256,), jnp.float32),
            pltpu.VMEM((_SL,), jnp.float32),
            pltpu.VMEM_SHARED((16, _NPAD), jnp.float32),
            pltpu.SemaphoreType.DMA,
            pltpu.SemaphoreType.DMA,
        ],
    )
    def kern(xl, xr, att, srcw, dstw, logits_o, mpart_o,
             idx_s, idx_d, rows_l, rows_r, att_v, logit_loc, m_loc,
             tbuf, red_out, stage, sem1, sem2):
        c = lax.axis_index("c")
        s = lax.axis_index("s")
        wid = s * 2 + c

        pltpu.sync_copy(att, att_v)

        neg = jnp.full((16,), -1e30, jnp.float32)

        def init_m(i, _):
            m_loc[pl.ds(i * 16, 16)] = neg
            return 0
        lax.fori_loop(0, _NPAD // 16, init_m, 0)

        att_ch = [att_v[pl.ds(16 * k, 16)] for k in range(HC)]
        iot = lax.iota(jnp.int32, 16)

        def batch_body(j, _):
            pltpu.sync_copy(srcw.at[wid, j], idx_s)
            pltpu.sync_copy(dstw.at[wid, j], idx_d)
            cp1 = pltpu.async_copy(xl.at[idx_s], rows_l, sem1)
            cp2 = pltpu.async_copy(xr.at[idx_d], rows_r, sem2)
            cp1.wait()
            cp2.wait()

            def group_body(g, _):
                for e in range(16):
                    acc = jnp.zeros((16,), jnp.float32)
                    for k in range(HC):
                        a = (rows_l[g * 16 + e, pl.ds(16 * k, 16)]
                             + rows_r[g * 16 + e, pl.ds(16 * k, 16)])
                        a = jnp.maximum(a, 0.2 * a)
                        acc = acc + a * att_ch[k]
                    tbuf[pl.ds(e * 16, 16)] = acc
                lg = jnp.zeros((16,), jnp.float32)
                for k in range(16):
                    lg = lg + plsc.load_gather(tbuf, [iot * 16 + k])
                ge = wid * _EPW + j * _B + g * 16 + iot
                lg = jnp.where(ge < _ETOT, lg,
                               jnp.full((16,), -1e30, jnp.float32))
                logit_loc[pl.ds(j * _B + g * 16, 16)] = lg
                dvec = idx_d[pl.ds(g * 16, 16)]
                for k in range(16):
                    mv = plsc.load_gather(m_loc, [dvec])
                    plsc.store_scatter(m_loc, [dvec], jnp.maximum(mv, lg),
                                       mask=iot == k)
                return 0
            lax.fori_loop(0, _B // 16, group_body, 0)
            return 0
        lax.fori_loop(0, _NB, batch_body, 0)

        pltpu.sync_copy(logit_loc, logits_o.at[wid])

        pltpu.sync_copy(m_loc, stage.at[s])
        plsc.subcore_barrier()
        for r in range(16):
            pltpu.sync_copy(stage.at[r, pl.ds(s * _SL, _SL)],
                            m_loc.at[pl.ds(r * _SL, _SL)])

        def red_body(i, _):
            acc = m_loc[pl.ds(16 * i, 16)]
            for r in range(1, 16):
                acc = jnp.maximum(acc, m_loc[pl.ds(r * _SL + 16 * i, 16)])
            red_out[pl.ds(16 * i, 16)] = acc
            return 0
        lax.fori_loop(0, _SL // 16, red_body, 0)
        pltpu.sync_copy(red_out, mpart_o.at[c, pl.ds(s * _SL, _SL)])

    return kern


def _make_kernel_b():
    @functools.partial(
        pl.kernel,
        mesh=_mesh,
        compiler_params=_params,
        out_type=[_f32(_NW, _EPW), _f32(2, _NPAD)],
        scratch_types=[
            pltpu.VMEM((_B,), jnp.int32),
            pltpu.VMEM((_EPW,), jnp.float32),
            pltpu.VMEM((_NPAD,), jnp.float32),
            pltpu.VMEM((_NPAD,), jnp.float32),
            pltpu.VMEM((_NPAD,), jnp.float32),
            pltpu.VMEM((_SL,), jnp.float32),
            pltpu.VMEM_SHARED((16, _NPAD), jnp.float32),
        ],
    )
    def kern(logits, dstw, mpart, ex_o, denpart_o,
             idx_d, lg_loc, mf, m2, den_loc, red_out, stage):
        c = lax.axis_index("c")
        s = lax.axis_index("s")
        wid = s * 2 + c

        pltpu.sync_copy(logits.at[wid], lg_loc)
        pltpu.sync_copy(mpart.at[0], mf)
        pltpu.sync_copy(mpart.at[1], m2)

        zero = jnp.zeros((16,), jnp.float32)

        def mmax(i, _):
            mf[pl.ds(16 * i, 16)] = jnp.maximum(mf[pl.ds(16 * i, 16)],
                                                m2[pl.ds(16 * i, 16)])
            den_loc[pl.ds(16 * i, 16)] = zero
            return 0
        lax.fori_loop(0, _NPAD // 16, mmax, 0)

        iot = lax.iota(jnp.int32, 16)

        def jbody(j, _):
            pltpu.sync_copy(dstw.at[wid, j], idx_d)

            def kbody(k, _):
                dvec = idx_d[pl.ds(16 * k, 16)]
                lgv = lg_loc[pl.ds(j * _B + 16 * k, 16)]
                mg = plsc.load_gather(mf, [dvec])
                exv = jnp.exp(lgv - mg)
                lg_loc[pl.ds(j * _B + 16 * k, 16)] = exv
                for q in range(16):
                    dv = plsc.load_gather(den_loc, [dvec])
                    plsc.store_scatter(den_loc, [dvec], dv + exv,
                                       mask=iot == q)
                return 0
            lax.fori_loop(0, _B // 16, kbody, 0)
            return 0
        lax.fori_loop(0, _NB, jbody, 0)

        pltpu.sync_copy(lg_loc, ex_o.at[wid])

        pltpu.sync_copy(den_loc, stage.at[s])
        plsc.subcore_barrier()
        for r in range(16):
            pltpu.sync_copy(stage.at[r, pl.ds(s * _SL, _SL)],
                            den_loc.at[pl.ds(r * _SL, _SL)])

        def red_body(i, _):
            acc = den_loc[pl.ds(16 * i, 16)]
            for r in range(1, 16):
                acc = acc + den_loc[pl.ds(r * _SL + 16 * i, 16)]
            red_out[pl.ds(16 * i, 16)] = acc
            return 0
        lax.fori_loop(0, _SL // 16, red_body, 0)
        pltpu.sync_copy(red_out, denpart_o.at[c, pl.ds(s * _SL, _SL)])

    return kern


def _make_kernel_c(halves):
    @functools.partial(
        pl.kernel,
        mesh=_mesh,
        compiler_params=_params,
        out_type=[_f32(2, halves, _NPAD, 128)],
        scratch_types=[
            pltpu.VMEM((_B,), jnp.int32),
            pltpu.VMEM((_B,), jnp.int32),
            pltpu.VMEM((_B,), jnp.float32),
            pltpu.VMEM((_NPAD,), jnp.float32),
            pltpu.VMEM((_NPAD,), jnp.float32),
            pltpu.VMEM((_B, 128), jnp.float32),
            pltpu.VMEM_SHARED((_NPAD, 128), jnp.float32),
            pltpu.SemaphoreType.DMA,
        ],
    )
    def kern(*args):
        (ex_i, srcw, dstw, denpart) = args[:4]
        xl_halves = args[4:4 + halves]
        outpart_o = args[4 + halves]
        (idx_s, idx_d, al_b, den_f, den2, rows, out_sh, sem) = \
            args[5 + halves:]

        c = lax.axis_index("c")
        s = lax.axis_index("s")
        wid = s * 2 + c

        pltpu.sync_copy(denpart.at[0], den_f)
        pltpu.sync_copy(denpart.at[1], den2)

        zero = jnp.zeros((16,), jnp.float32)

        def dsum(i, _):
            den_f[pl.ds(16 * i, 16)] = (den_f[pl.ds(16 * i, 16)]
                                        + den2[pl.ds(16 * i, 16)] + 1e-16)
            return 0
        lax.fori_loop(0, _NPAD // 16, dsum, 0)

        for half in range(halves):
            xlh = xl_halves[half]

            for e in range(16):
                for k in range(8):
                    rows[e, pl.ds(16 * k, 16)] = zero

            def zrow(i, _):
                pltpu.sync_copy(rows.at[pl.ds(0, 16), :],
                                out_sh.at[pl.ds(s * _SL + i * 16, 16), :])
                return 0
            lax.fori_loop(0, _SL // 16, zrow, 0)
            plsc.subcore_barrier()

            def batch_body(j, _):
                pltpu.sync_copy(srcw.at[wid, j], idx_s)
                pltpu.sync_copy(dstw.at[wid, j], idx_d)
                pltpu.sync_copy(ex_i.at[wid, pl.ds(j * _B, _B)], al_b)
                pltpu.async_copy(xlh.at[idx_s], rows, sem).wait()

                def group_body(g, _):
                    dvec = idx_d[pl.ds(g * 16, 16)]
                    exv = al_b[pl.ds(g * 16, 16)]
                    dg = plsc.load_gather(den_f, [dvec])
                    alv = exv / dg
                    for e in range(16):
                        a = alv[e]
                        for k in range(8):
                            rows[g * 16 + e, pl.ds(16 * k, 16)] = (
                                rows[g * 16 + e, pl.ds(16 * k, 16)] * a)
                    return 0
                lax.fori_loop(0, _B // 16, group_body, 0)
                pltpu.sync_copy(rows, out_sh.at[idx_d], add=True)
                return 0
            lax.fori_loop(0, _NB, batch_body, 0)
            plsc.subcore_barrier()
            pltpu.sync_copy(out_sh.at[pl.ds(s * _SL, _SL), :],
                            outpart_o.at[c, half, pl.ds(s * _SL, _SL), :])
            plsc.subcore_barrier()

    return kern


_kernel_a256 = _make_kernel_a(256)
_kernel_a128 = _make_kernel_a(128)
_kernel_b = _make_kernel_b()
_kernel_c2 = _make_kernel_c(2)
_kernel_c1 = _make_kernel_c(1)


def _tc0_body(x_ref, wl_ref, wr_ref, xl_ref, xr_ref):
    xl_ref[...] = jnp.dot(x_ref[...], wl_ref[...],
                          preferred_element_type=jnp.float32)
    xr_ref[...] = jnp.dot(x_ref[...], wr_ref[...],
                          preferred_element_type=jnp.float32)


def _tc1_body(op_ref, b1_ref, g_ref, bb_ref, w2l_ref, w2r_ref, w3l_ref,
              w3r_ref, xl2_ref, xr2_ref, xl3_ref, xr3_ref):
    h = jnp.concatenate(
        [op_ref[0, 0, :_N, :] + op_ref[1, 0, :_N, :],
         op_ref[0, 1, :_N, :] + op_ref[1, 1, :_N, :]], axis=1)
    h = h + b1_ref[...]
    mean = jnp.mean(h, axis=0, keepdims=True)
    var = jnp.mean((h - mean) ** 2, axis=0, keepdims=True)
    h = g_ref[...] * (h - mean) / jnp.sqrt(var + 1e-5) + bb_ref[...]
    h = jnp.maximum(h, 0.0)
    xl2_ref[...] = jnp.dot(h, w2l_ref[...], preferred_element_type=jnp.float32)
    xr2_ref[...] = jnp.dot(h, w2r_ref[...], preferred_element_type=jnp.float32)
    xl3_ref[...] = jnp.dot(h, w3l_ref[...], preferred_element_type=jnp.float32)
    xr3_ref[...] = jnp.dot(h, w3r_ref[...], preferred_element_type=jnp.float32)


def _tc2_body(p2_ref, p3_ref, b2_ref, b3_ref, mu_ref, lv_ref):
    mu_ref[...] = p2_ref[0, 0, :_N, :] + p2_ref[1, 0, :_N, :] + b2_ref[...]
    lv_ref[...] = p3_ref[0, 0, :_N, :] + p3_ref[1, 0, :_N, :] + b3_ref[...]


def _gat_layer(xl, xr, att, srcw, dstw, kern_a, kern_c, halves):
    logits, mpart = kern_a(xl, xr, att, srcw, dstw)
    ex, denpart = _kernel_b(logits, dstw, mpart)
    xl_halves = [xl[:, 128 * h:128 * (h + 1)] for h in range(halves)]
    (outpart,) = kern_c(ex, srcw, dstw, denpart, *xl_halves)
    return outpart


def kernel(x, adj, W1l, W1r, att1, b1, bn_g, bn_b, W2l, W2r, att2, b2,
           W3l, W3r, att3, b3):
    loop = jnp.arange(_N, dtype=adj.dtype)
    pad = jnp.zeros((_NW * _EPW - _ETOT,), adj.dtype)
    srcw = jnp.concatenate([adj[0], loop, pad]).reshape(_NW, _NB, _B)
    dstw = jnp.concatenate([adj[1], loop, pad]).reshape(_NW, _NB, _B)

    xl1, xr1 = pl.pallas_call(
        _tc0_body,
        out_shape=[_f32(_N, 256), _f32(_N, 256)],
    )(x, W1l, W1r)

    op1 = _gat_layer(xl1, xr1, att1, srcw, dstw, _kernel_a256, _kernel_c2, 2)

    xl2, xr2, xl3, xr3 = pl.pallas_call(
        _tc1_body,
        out_shape=[_f32(_N, 128)] * 4,
    )(op1, b1, bn_g, bn_b, W2l, W2r, W3l, W3r)

    op2 = _gat_layer(xl2, xr2, att2, srcw, dstw, _kernel_a128, _kernel_c1, 1)
    op3 = _gat_layer(xl3, xr3, att3, srcw, dstw, _kernel_a128, _kernel_c1, 1)

    mu, logvar = pl.pallas_call(
        _tc2_body,
        out_shape=[_f32(_N, 128), _f32(_N, 128)],
    )(op2, op3, b2, b3)
    return (mu, logvar)


# speedup vs baseline: 6.5606x; 1.1854x over previous
import functools
import jax
import jax.numpy as jnp
from jax import lax
from jax.experimental import pallas as pl
from jax.experimental.pallas import tpu as pltpu
from jax.experimental.pallas import tpu_sc as plsc

_N = 10000
_E = 320000
_ETOT = _E + _N
_NW = 32
_B = 64
_NB = -(-_ETOT // (_NW * _B))
_EPW = _NB * _B
_NPAD = 10240
_SL = _NPAD // 16

_mesh = plsc.VectorSubcoreMesh(core_axis_name="c", subcore_axis_name="s",
                               num_cores=2, num_subcores=16)
_params = pltpu.CompilerParams(needs_layout_passes=False)


def _f32(*shape):
    return jax.ShapeDtypeStruct(shape, jnp.float32)


def _make_kernel_a(H):
    HC = H // 16

    @functools.partial(
        pl.kernel,
        mesh=_mesh,
        compiler_params=_params,
        out_type=[_f32(_NW, _EPW), _f32(2, _NPAD)],
        scratch_types=[
            pltpu.VMEM((_B,), jnp.int32),
            pltpu.VMEM((_B,), jnp.int32),
            pltpu.VMEM((_B,), jnp.int32),
            pltpu.VMEM((_B,), jnp.int32),
            pltpu.VMEM((_B, H), jnp.float32),
            pltpu.VMEM((_B, H), jnp.float32),
            pltpu.VMEM((_B, H), jnp.float32),
            pltpu.VMEM((_B, H), jnp.float32),
            pltpu.VMEM((H,), jnp.float32),
            pltpu.VMEM((_EPW,), jnp.float32),
            pltpu.VMEM((_NPAD,), jnp.float32),
            pltpu.VMEM((256,), jnp.float32),
            pltpu.VMEM((_SL,), jnp.float32),
            pltpu.VMEM_SHARED((16, _NPAD), jnp.float32),
            pltpu.SemaphoreType.DMA,
            pltpu.SemaphoreType.DMA,
            pltpu.SemaphoreType.DMA,
            pltpu.SemaphoreType.DMA,
        ],
    )
    def kern(xl, xr, att, srcw, dstw, logits_o, mpart_o,
             idx_s0, idx_s1, idx_d0, idx_d1, rows_l0, rows_l1,
             rows_r0, rows_r1, att_v, logit_loc, m_loc,
             tbuf, red_out, stage, sl0, sl1, sr0, sr1):
        c = lax.axis_index("c")
        s = lax.axis_index("s")
        wid = s * 2 + c
        slots = ((idx_s0, idx_d0, rows_l0, rows_r0, sl0, sr0),
                 (idx_s1, idx_d1, rows_l1, rows_r1, sl1, sr1))

        pltpu.sync_copy(att, att_v)

        neg = jnp.full((16,), -1e30, jnp.float32)

        def init_m(i, _):
            m_loc[pl.ds(i * 16, 16)] = neg
            return 0
        lax.fori_loop(0, _NPAD // 16, init_m, 0)

        att_ch = [att_v[pl.ds(16 * k, 16)] for k in range(HC)]
        iot = lax.iota(jnp.int32, 16)

        def fire(j, slot):
            idx_s, idx_d, rows_l, rows_r, sem_l, sem_r = slot
            pltpu.sync_copy(srcw.at[wid, j], idx_s)
            pltpu.sync_copy(dstw.at[wid, j], idx_d)
            pltpu.async_copy(xl.at[idx_s], rows_l, sem_l)
            pltpu.async_copy(xr.at[idx_d], rows_r, sem_r)

        def process(j, slot):
            idx_s, idx_d, rows_l, rows_r, sem_l, sem_r = slot
            pltpu.make_async_copy(xl.at[idx_s], rows_l, sem_l).wait()
            pltpu.make_async_copy(xr.at[idx_d], rows_r, sem_r).wait()

            def group_body(g, _):
                for e in range(16):
                    acc = jnp.zeros((16,), jnp.float32)
                    for k in range(HC):
                        a = (rows_l[g * 16 + e, pl.ds(16 * k, 16)]
                             + rows_r[g * 16 + e, pl.ds(16 * k, 16)])
                        a = jnp.maximum(a, 0.2 * a)
                        acc = acc + a * att_ch[k]
                    tbuf[pl.ds(e * 16, 16)] = acc
                lg = jnp.zeros((16,), jnp.float32)
                for k in range(16):
                    lg = lg + plsc.load_gather(tbuf, [iot * 16 + k])
                ge = wid * _EPW + j * _B + g * 16 + iot
                lg = jnp.where(ge < _ETOT, lg,
                               jnp.full((16,), -1e30, jnp.float32))
                logit_loc[pl.ds(j * _B + g * 16, 16)] = lg
                dvec = idx_d[pl.ds(g * 16, 16)]
                for k in range(16):
                    mv = plsc.load_gather(m_loc, [dvec])
                    plsc.store_scatter(m_loc, [dvec], jnp.maximum(mv, lg),
                                       mask=iot == k)
                return 0
            lax.fori_loop(0, _B // 16, group_body, 0)

        fire(0, slots[0])
        fire(1, slots[1])

        def pipe_body(i, _):
            for p in range(2):
                j = 2 * i + p
                process(j, slots[p])

                @pl.when(j + 2 < _NB)
                def _():
                    fire(j + 2, slots[p])
            return 0
        lax.fori_loop(0, _NB // 2, pipe_body, 0)

        pltpu.sync_copy(logit_loc, logits_o.at[wid])

        pltpu.sync_copy(m_loc, stage.at[s])
        plsc.subcore_barrier()
        for r in range(16):
            pltpu.sync_copy(stage.at[r, pl.ds(s * _SL, _SL)],
                            m_loc.at[pl.ds(r * _SL, _SL)])

        def red_body(i, _):
            acc = m_loc[pl.ds(16 * i, 16)]
            for r in range(1, 16):
                acc = jnp.maximum(acc, m_loc[pl.ds(r * _SL + 16 * i, 16)])
            red_out[pl.ds(16 * i, 16)] = acc
            return 0
        lax.fori_loop(0, _SL // 16, red_body, 0)
        pltpu.sync_copy(red_out, mpart_o.at[c, pl.ds(s * _SL, _SL)])

    return kern


def _make_kernel_b():
    @functools.partial(
        pl.kernel,
        mesh=_mesh,
        compiler_params=_params,
        out_type=[_f32(_NW, _EPW), _f32(2, _NPAD)],
        scratch_types=[
            pltpu.VMEM((_NB, _B), jnp.int32),
            pltpu.VMEM((_EPW,), jnp.float32),
            pltpu.VMEM((_NPAD,), jnp.float32),
            pltpu.VMEM((_NPAD,), jnp.float32),
            pltpu.VMEM((_NPAD,), jnp.float32),
            pltpu.VMEM((_SL,), jnp.float32),
            pltpu.VMEM_SHARED((16, _NPAD), jnp.float32),
        ],
    )
    def kern(logits, dstw, mpart, ex_o, denpart_o,
             dst_loc, lg_loc, mf, m2, den_loc, red_out, stage):
        c = lax.axis_index("c")
        s = lax.axis_index("s")
        wid = s * 2 + c

        pltpu.sync_copy(dstw.at[wid], dst_loc)
        pltpu.sync_copy(logits.at[wid], lg_loc)
        pltpu.sync_copy(mpart.at[0], mf)
        pltpu.sync_copy(mpart.at[1], m2)

        zero = jnp.zeros((16,), jnp.float32)

        def mmax(i, _):
            mf[pl.ds(16 * i, 16)] = jnp.maximum(mf[pl.ds(16 * i, 16)],
                                                m2[pl.ds(16 * i, 16)])
            den_loc[pl.ds(16 * i, 16)] = zero
            return 0
        lax.fori_loop(0, _NPAD // 16, mmax, 0)

        iot = lax.iota(jnp.int32, 16)

        def jbody(j, _):
            def kbody(k, _):
                dvec = dst_loc[j, pl.ds(16 * k, 16)]
                lgv = lg_loc[pl.ds(j * _B + 16 * k, 16)]
                mg = plsc.load_gather(mf, [dvec])
                exv = jnp.exp(lgv - mg)
                lg_loc[pl.ds(j * _B + 16 * k, 16)] = exv
                for q in range(16):
                    dv = plsc.load_gather(den_loc, [dvec])
                    plsc.store_scatter(den_loc, [dvec], dv + exv,
                                       mask=iot == q)
                return 0
            lax.fori_loop(0, _B // 16, kbody, 0)
            return 0
        lax.fori_loop(0, _NB, jbody, 0)

        pltpu.sync_copy(lg_loc, ex_o.at[wid])

        pltpu.sync_copy(den_loc, stage.at[s])
        plsc.subcore_barrier()
        for r in range(16):
            pltpu.sync_copy(stage.at[r, pl.ds(s * _SL, _SL)],
                            den_loc.at[pl.ds(r * _SL, _SL)])

        def red_body(i, _):
            acc = den_loc[pl.ds(16 * i, 16)]
            for r in range(1, 16):
                acc = acc + den_loc[pl.ds(r * _SL + 16 * i, 16)]
            red_out[pl.ds(16 * i, 16)] = acc
            return 0
        lax.fori_loop(0, _SL // 16, red_body, 0)
        pltpu.sync_copy(red_out, denpart_o.at[c, pl.ds(s * _SL, _SL)])

    return kern


def _make_kernel_c(halves):
    @functools.partial(
        pl.kernel,
        mesh=_mesh,
        compiler_params=_params,
        out_type=[_f32(2, halves, _NPAD, 128)],
        scratch_types=[
            pltpu.VMEM((_B,), jnp.int32),
            pltpu.VMEM((_B,), jnp.int32),
            pltpu.VMEM((_B,), jnp.int32),
            pltpu.VMEM((_B,), jnp.int32),
            pltpu.VMEM((_B,), jnp.float32),
            pltpu.VMEM((_B,), jnp.float32),
            pltpu.VMEM((_NPAD,), jnp.float32),
            pltpu.VMEM((_NPAD,), jnp.float32),
            pltpu.VMEM((_B, 128), jnp.float32),
            pltpu.VMEM((_B, 128), jnp.float32),
            pltpu.VMEM_SHARED((_NPAD, 128), jnp.float32),
            pltpu.SemaphoreType.DMA,
            pltpu.SemaphoreType.DMA,
        ],
    )
    def kern(*args):
        (ex_i, srcw, dstw, denpart) = args[:4]
        xl_halves = args[4:4 + halves]
        outpart_o = args[4 + halves]
        (idx_s0, idx_s1, idx_d0, idx_d1, ex0, ex1, den_f, den2,
         rows0, rows1, out_sh, sem0, sem1) = args[5 + halves:]

        c = lax.axis_index("c")
        s = lax.axis_index("s")
        wid = s * 2 + c
        slots = ((idx_s0, idx_d0, ex0, rows0, sem0),
                 (idx_s1, idx_d1, ex1, rows1, sem1))

        pltpu.sync_copy(denpart.at[0], den_f)
        pltpu.sync_copy(denpart.at[1], den2)

        zero = jnp.zeros((16,), jnp.float32)

        def dsum(i, _):
            den_f[pl.ds(16 * i, 16)] = (den_f[pl.ds(16 * i, 16)]
                                        + den2[pl.ds(16 * i, 16)] + 1e-16)
            return 0
        lax.fori_loop(0, _NPAD // 16, dsum, 0)

        for half in range(halves):
            xlh = xl_halves[half]

            def fire(j, slot):
                idx_s, idx_d, exb, rows, sem = slot
                pltpu.sync_copy(srcw.at[wid, j], idx_s)
                pltpu.sync_copy(dstw.at[wid, j], idx_d)
                pltpu.sync_copy(ex_i.at[wid, pl.ds(j * _B, _B)], exb)
                pltpu.async_copy(xlh.at[idx_s], rows, sem)

            def process(j, slot):
                idx_s, idx_d, exb, rows, sem = slot
                pltpu.make_async_copy(xlh.at[idx_s], rows, sem).wait()

                def group_body(g, _):
                    dvec = idx_d[pl.ds(g * 16, 16)]
                    exv = exb[pl.ds(g * 16, 16)]
                    dg = plsc.load_gather(den_f, [dvec])
                    alv = exv / dg
                    for e in range(16):
                        a = alv[e]
                        for k in range(8):
                            rows[g * 16 + e, pl.ds(16 * k, 16)] = (
                                rows[g * 16 + e, pl.ds(16 * k, 16)] * a)
                    return 0
                lax.fori_loop(0, _B // 16, group_body, 0)
                pltpu.sync_copy(rows, out_sh.at[idx_d], add=True)

            for e in range(16):
                for k in range(8):
                    rows0[e, pl.ds(16 * k, 16)] = zero

            def zrow(i, _):
                pltpu.sync_copy(rows0.at[pl.ds(0, 16), :],
                                out_sh.at[pl.ds(s * _SL + i * 16, 16), :])
                return 0
            lax.fori_loop(0, _SL // 16, zrow, 0)
            plsc.subcore_barrier()

            fire(0, slots[0])
            fire(1, slots[1])

            def pipe_body(i, _):
                for p in range(2):
                    j = 2 * i + p
                    process(j, slots[p])

                    @pl.when(j + 2 < _NB)
                    def _():
                        fire(j + 2, slots[p])
                return 0
            lax.fori_loop(0, _NB // 2, pipe_body, 0)
            plsc.subcore_barrier()
            pltpu.sync_copy(out_sh.at[pl.ds(s * _SL, _SL), :],
                            outpart_o.at[c, half, pl.ds(s * _SL, _SL), :])
            plsc.subcore_barrier()

    return kern


_kernel_a256 = _make_kernel_a(256)
_kernel_a128 = _make_kernel_a(128)
_kernel_b = _make_kernel_b()
_kernel_c2 = _make_kernel_c(2)
_kernel_c1 = _make_kernel_c(1)


def _tc0_body(x_ref, wl_ref, wr_ref, xl_ref, xr_ref):
    xl_ref[...] = jnp.dot(x_ref[...], wl_ref[...],
                          preferred_element_type=jnp.float32)
    xr_ref[...] = jnp.dot(x_ref[...], wr_ref[...],
                          preferred_element_type=jnp.float32)


def _tc1_body(op_ref, b1_ref, g_ref, bb_ref, w2l_ref, w2r_ref, w3l_ref,
              w3r_ref, xl2_ref, xr2_ref, xl3_ref, xr3_ref):
    h = jnp.concatenate(
        [op_ref[0, 0, :_N, :] + op_ref[1, 0, :_N, :],
         op_ref[0, 1, :_N, :] + op_ref[1, 1, :_N, :]], axis=1)
    h = h + b1_ref[...]
    mean = jnp.mean(h, axis=0, keepdims=True)
    var = jnp.mean((h - mean) ** 2, axis=0, keepdims=True)
    h = g_ref[...] * (h - mean) / jnp.sqrt(var + 1e-5) + bb_ref[...]
    h = jnp.maximum(h, 0.0)
    xl2_ref[...] = jnp.dot(h, w2l_ref[...], preferred_element_type=jnp.float32)
    xr2_ref[...] = jnp.dot(h, w2r_ref[...], preferred_element_type=jnp.float32)
    xl3_ref[...] = jnp.dot(h, w3l_ref[...], preferred_element_type=jnp.float32)
    xr3_ref[...] = jnp.dot(h, w3r_ref[...], preferred_element_type=jnp.float32)


def _tc2_body(p2_ref, p3_ref, b2_ref, b3_ref, mu_ref, lv_ref):
    mu_ref[...] = p2_ref[0, 0, :_N, :] + p2_ref[1, 0, :_N, :] + b2_ref[...]
    lv_ref[...] = p3_ref[0, 0, :_N, :] + p3_ref[1, 0, :_N, :] + b3_ref[...]


def _gat_layer(xl, xr, att, srcw, dstw, kern_a, kern_c, halves):
    logits, mpart = kern_a(xl, xr, att, srcw, dstw)
    ex, denpart = _kernel_b(logits, dstw, mpart)
    xl_halves = [xl[:, 128 * h:128 * (h + 1)] for h in range(halves)]
    (outpart,) = kern_c(ex, srcw, dstw, denpart, *xl_halves)
    return outpart


def kernel(x, adj, W1l, W1r, att1, b1, bn_g, bn_b, W2l, W2r, att2, b2,
           W3l, W3r, att3, b3):
    loop = jnp.arange(_N, dtype=adj.dtype)
    pad = jnp.zeros((_NW * _EPW - _ETOT,), adj.dtype)
    srcw = jnp.concatenate([adj[0], loop, pad]).reshape(_NW, _NB, _B)
    dstw = jnp.concatenate([adj[1], loop, pad]).reshape(_NW, _NB, _B)

    xl1, xr1 = pl.pallas_call(
        _tc0_body,
        out_shape=[_f32(_N, 256), _f32(_N, 256)],
    )(x, W1l, W1r)

    op1 = _gat_layer(xl1, xr1, att1, srcw, dstw, _kernel_a256, _kernel_c2, 2)

    xl2, xr2, xl3, xr3 = pl.pallas_call(
        _tc1_body,
        out_shape=[_f32(_N, 128)] * 4,
    )(op1, b1, bn_g, bn_b, W2l, W2r, W3l, W3r)

    op2 = _gat_layer(xl2, xr2, att2, srcw, dstw, _kernel_a128, _kernel_c1, 1)
    op3 = _gat_layer(xl3, xr3, att3, srcw, dstw, _kernel_a128, _kernel_c1, 1)

    mu, logvar = pl.pallas_call(
        _tc2_body,
        out_shape=[_f32(_N, 128), _f32(_N, 128)],
    )(op2, op3, b2, b3)
    return (mu, logvar)


# speedup vs baseline: 7.0918x; 1.0810x over previous
import functools
import jax
import jax.numpy as jnp
from jax import lax
from jax.experimental import pallas as pl
from jax.experimental.pallas import tpu as pltpu
from jax.experimental.pallas import tpu_sc as plsc

_N = 10000
_E = 320000
_ETOT = _E + _N
_NW = 32
_B = 64
_NB = -(-_ETOT // (_NW * _B))
_EPW = _NB * _B
_NPAD = 10240
_SL = _NPAD // 16

_mesh = plsc.VectorSubcoreMesh(core_axis_name="c", subcore_axis_name="s",
                               num_cores=2, num_subcores=16)
_params = pltpu.CompilerParams(needs_layout_passes=False)


def _f32(*shape):
    return jax.ShapeDtypeStruct(shape, jnp.float32)


def _make_kernel_a(H):
    HC = H // 16

    @functools.partial(
        pl.kernel,
        mesh=_mesh,
        compiler_params=_params,
        out_type=[_f32(_NW, _EPW), _f32(2, _NPAD)],
        scratch_types=[
            pltpu.VMEM((_B,), jnp.int32),
            pltpu.VMEM((_B,), jnp.int32),
            pltpu.VMEM((_B,), jnp.int32),
            pltpu.VMEM((_B,), jnp.int32),
            pltpu.VMEM((_B, H), jnp.float32),
            pltpu.VMEM((_B, H), jnp.float32),
            pltpu.VMEM((_B, H), jnp.float32),
            pltpu.VMEM((_B, H), jnp.float32),
            pltpu.VMEM((H,), jnp.float32),
            pltpu.VMEM((_EPW,), jnp.float32),
            pltpu.VMEM((_NPAD,), jnp.float32),
            pltpu.VMEM((256,), jnp.float32),
            pltpu.VMEM((_SL,), jnp.float32),
            pltpu.VMEM_SHARED((16, _NPAD), jnp.float32),
            pltpu.SemaphoreType.DMA,
            pltpu.SemaphoreType.DMA,
            pltpu.SemaphoreType.DMA,
            pltpu.SemaphoreType.DMA,
        ],
    )
    def kern(xl, xr, att, srcw, dstw, logits_o, mpart_o,
             idx_s0, idx_s1, idx_d0, idx_d1, rows_l0, rows_l1,
             rows_r0, rows_r1, att_v, logit_loc, m_loc,
             tbuf, red_out, stage, sl0, sl1, sr0, sr1):
        c = lax.axis_index("c")
        s = lax.axis_index("s")
        wid = s * 2 + c
        slots = ((idx_s0, idx_d0, rows_l0, rows_r0, sl0, sr0),
                 (idx_s1, idx_d1, rows_l1, rows_r1, sl1, sr1))

        pltpu.sync_copy(att, att_v)

        neg = jnp.full((16,), -1e30, jnp.float32)

        def init_m(i, _):
            m_loc[pl.ds(i * 16, 16)] = neg
            return 0
        lax.fori_loop(0, _NPAD // 16, init_m, 0)

        att_ch = [att_v[pl.ds(16 * k, 16)] for k in range(HC)]
        iot = lax.iota(jnp.int32, 16)

        def fire(j, slot):
            idx_s, idx_d, rows_l, rows_r, sem_l, sem_r = slot
            pltpu.sync_copy(srcw.at[wid, j], idx_s)
            pltpu.sync_copy(dstw.at[wid, j], idx_d)
            pltpu.async_copy(xl.at[idx_s], rows_l, sem_l)
            pltpu.async_copy(xr.at[idx_d], rows_r, sem_r)

        def process(j, slot):
            idx_s, idx_d, rows_l, rows_r, sem_l, sem_r = slot
            pltpu.make_async_copy(xl.at[idx_s], rows_l, sem_l).wait()
            pltpu.make_async_copy(xr.at[idx_d], rows_r, sem_r).wait()

            def group_body(g, _):
                for e in range(16):
                    acc = jnp.zeros((16,), jnp.float32)
                    for k in range(HC):
                        a = (rows_l[g * 16 + e, pl.ds(16 * k, 16)]
                             + rows_r[g * 16 + e, pl.ds(16 * k, 16)])
                        a = jnp.maximum(a, 0.2 * a)
                        acc = acc + a * att_ch[k]
                    tbuf[pl.ds(e * 16, 16)] = acc
                lg = jnp.zeros((16,), jnp.float32)
                for k in range(16):
                    lg = lg + plsc.load_gather(tbuf, [iot * 16 + k])
                ge = wid * _EPW + j * _B + g * 16 + iot
                lg = jnp.where(ge < _ETOT, lg,
                               jnp.full((16,), -1e30, jnp.float32))
                logit_loc[pl.ds(j * _B + g * 16, 16)] = lg
                dvec = idx_d[pl.ds(g * 16, 16)]
                for k in range(16):
                    mv = plsc.load_gather(m_loc, [dvec])
                    plsc.store_scatter(m_loc, [dvec], jnp.maximum(mv, lg),
                                       mask=iot == k)
                return 0
            lax.fori_loop(0, _B // 16, group_body, 0)

        fire(0, slots[0])
        fire(1, slots[1])

        def pipe_body(i, _):
            for p in range(2):
                j = 2 * i + p
                process(j, slots[p])

                @pl.when(j + 2 < _NB)
                def _():
                    fire(j + 2, slots[p])
            return 0
        lax.fori_loop(0, _NB // 2, pipe_body, 0)

        pltpu.sync_copy(logit_loc, logits_o.at[wid])

        pltpu.sync_copy(m_loc, stage.at[s])
        plsc.subcore_barrier()
        for r in range(16):
            pltpu.sync_copy(stage.at[r, pl.ds(s * _SL, _SL)],
                            m_loc.at[pl.ds(r * _SL, _SL)])

        def red_body(i, _):
            acc = m_loc[pl.ds(16 * i, 16)]
            for r in range(1, 16):
                acc = jnp.maximum(acc, m_loc[pl.ds(r * _SL + 16 * i, 16)])
            red_out[pl.ds(16 * i, 16)] = acc
            return 0
        lax.fori_loop(0, _SL // 16, red_body, 0)
        pltpu.sync_copy(red_out, mpart_o.at[c, pl.ds(s * _SL, _SL)])

    return kern


def _make_kernel_c(halves):
    @functools.partial(
        pl.kernel,
        mesh=_mesh,
        compiler_params=_params,
        out_type=[_f32(2, halves, _NPAD, 128), _f32(2, _NPAD)],
        scratch_types=[
            pltpu.VMEM((_B,), jnp.int32),
            pltpu.VMEM((_B,), jnp.int32),
            pltpu.VMEM((_B,), jnp.int32),
            pltpu.VMEM((_B,), jnp.int32),
            pltpu.VMEM((_B,), jnp.float32),
            pltpu.VMEM((_B,), jnp.float32),
            pltpu.VMEM((_NPAD,), jnp.float32),
            pltpu.VMEM((_NPAD,), jnp.float32),
            pltpu.VMEM((_B, 128), jnp.float32),
            pltpu.VMEM((_B, 128), jnp.float32),
            pltpu.VMEM_SHARED((_NPAD, 128), jnp.float32),
            pltpu.VMEM_SHARED((_NPAD,), jnp.float32),
            pltpu.SemaphoreType.DMA,
            pltpu.SemaphoreType.DMA,
        ],
    )
    def kern(*args):
        (lg_i, srcw, dstw, mpart) = args[:4]
        xl_halves = args[4:4 + halves]
        outpart_o = args[4 + halves]
        denpart_o = args[5 + halves]
        (idx_s0, idx_s1, idx_d0, idx_d1, ex0, ex1, mf, m2,
         rows0, rows1, out_sh, den_sh, sem0, sem1) = args[6 + halves:]

        c = lax.axis_index("c")
        s = lax.axis_index("s")
        wid = s * 2 + c
        slots = ((idx_s0, idx_d0, ex0, rows0, sem0),
                 (idx_s1, idx_d1, ex1, rows1, sem1))

        pltpu.sync_copy(mpart.at[0], mf)
        pltpu.sync_copy(mpart.at[1], m2)

        zero = jnp.zeros((16,), jnp.float32)

        def mmax(i, _):
            mf[pl.ds(16 * i, 16)] = jnp.maximum(mf[pl.ds(16 * i, 16)],
                                                m2[pl.ds(16 * i, 16)])
            return 0
        lax.fori_loop(0, _NPAD // 16, mmax, 0)

        for half in range(halves):
            xlh = xl_halves[half]

            def fire(j, slot):
                idx_s, idx_d, exb, rows, sem = slot
                pltpu.sync_copy(srcw.at[wid, j], idx_s)
                pltpu.sync_copy(dstw.at[wid, j], idx_d)
                pltpu.sync_copy(lg_i.at[wid, pl.ds(j * _B, _B)], exb)
                pltpu.async_copy(xlh.at[idx_s], rows, sem)

            def process(j, slot):
                idx_s, idx_d, exb, rows, sem = slot
                pltpu.make_async_copy(xlh.at[idx_s], rows, sem).wait()

                def group_body(g, _):
                    dvec = idx_d[pl.ds(g * 16, 16)]
                    lgv = exb[pl.ds(g * 16, 16)]
                    mg = plsc.load_gather(mf, [dvec])
                    alv = jnp.exp(lgv - mg)
                    if half == 0:
                        exb[pl.ds(g * 16, 16)] = alv
                    for e in range(16):
                        a = alv[e]
                        for k in range(8):
                            rows[g * 16 + e, pl.ds(16 * k, 16)] = (
                                rows[g * 16 + e, pl.ds(16 * k, 16)] * a)
                    return 0
                lax.fori_loop(0, _B // 16, group_body, 0)
                pltpu.sync_copy(rows, out_sh.at[idx_d], add=True)
                if half == 0:
                    pltpu.sync_copy(exb, den_sh.at[idx_d], add=True)

            for e in range(16):
                for k in range(8):
                    rows0[e, pl.ds(16 * k, 16)] = zero

            def zrow(i, _):
                pltpu.sync_copy(rows0.at[pl.ds(0, 16), :],
                                out_sh.at[pl.ds(s * _SL + i * 16, 16), :])
                return 0
            lax.fori_loop(0, _SL // 16, zrow, 0)
            if half == 0:
                def zden(i, _):
                    pltpu.sync_copy(rows0.at[0, pl.ds(0, 16)],
                                    den_sh.at[pl.ds(s * _SL + i * 16, 16)])
                    return 0
                lax.fori_loop(0, _SL // 16, zden, 0)
            plsc.subcore_barrier()

            fire(0, slots[0])
            fire(1, slots[1])

            def pipe_body(i, _):
                for p in range(2):
                    j = 2 * i + p
                    process(j, slots[p])

                    @pl.when(j + 2 < _NB)
                    def _():
                        fire(j + 2, slots[p])
                return 0
            lax.fori_loop(0, _NB // 2, pipe_body, 0)
            plsc.subcore_barrier()
            pltpu.sync_copy(out_sh.at[pl.ds(s * _SL, _SL), :],
                            outpart_o.at[c, half, pl.ds(s * _SL, _SL), :])
            if half == 0:
                pltpu.sync_copy(den_sh.at[pl.ds(s * _SL, _SL)],
                                denpart_o.at[c, pl.ds(s * _SL, _SL)])
            plsc.subcore_barrier()

    return kern


_kernel_a256 = _make_kernel_a(256)
_kernel_a128 = _make_kernel_a(128)
_kernel_c2 = _make_kernel_c(2)
_kernel_c1 = _make_kernel_c(1)


def _tc0_body(x_ref, wl_ref, wr_ref, xl_ref, xr_ref):
    xl_ref[...] = jnp.dot(x_ref[...], wl_ref[...],
                          preferred_element_type=jnp.float32)
    xr_ref[...] = jnp.dot(x_ref[...], wr_ref[...],
                          preferred_element_type=jnp.float32)


def _tc1_body(op_ref, den_ref, b1_ref, g_ref, bb_ref, w2l_ref, w2r_ref,
              w3l_ref, w3r_ref, xl2_ref, xr2_ref, xl3_ref, xr3_ref):
    den = (den_ref[0, :_N] + den_ref[1, :_N] + 1e-16)[:, None]
    h = jnp.concatenate(
        [op_ref[0, 0, :_N, :] + op_ref[1, 0, :_N, :],
         op_ref[0, 1, :_N, :] + op_ref[1, 1, :_N, :]], axis=1)
    h = h / den + b1_ref[...]
    mean = jnp.mean(h, axis=0, keepdims=True)
    var = jnp.mean((h - mean) ** 2, axis=0, keepdims=True)
    h = g_ref[...] * (h - mean) / jnp.sqrt(var + 1e-5) + bb_ref[...]
    h = jnp.maximum(h, 0.0)
    xl2_ref[...] = jnp.dot(h, w2l_ref[...], preferred_element_type=jnp.float32)
    xr2_ref[...] = jnp.dot(h, w2r_ref[...], preferred_element_type=jnp.float32)
    xl3_ref[...] = jnp.dot(h, w3l_ref[...], preferred_element_type=jnp.float32)
    xr3_ref[...] = jnp.dot(h, w3r_ref[...], preferred_element_type=jnp.float32)


def _tc2_body(p2_ref, d2_ref, p3_ref, d3_ref, b2_ref, b3_ref, mu_ref,
              lv_ref):
    den2 = (d2_ref[0, :_N] + d2_ref[1, :_N] + 1e-16)[:, None]
    den3 = (d3_ref[0, :_N] + d3_ref[1, :_N] + 1e-16)[:, None]
    mu_ref[...] = ((p2_ref[0, 0, :_N, :] + p2_ref[1, 0, :_N, :]) / den2
                   + b2_ref[...])
    lv_ref[...] = ((p3_ref[0, 0, :_N, :] + p3_ref[1, 0, :_N, :]) / den3
                   + b3_ref[...])


def _gat_layer(xl, xr, att, srcw, dstw, kern_a, kern_c, halves):
    logits, mpart = kern_a(xl, xr, att, srcw, dstw)
    xl_halves = [xl[:, 128 * h:128 * (h + 1)] for h in range(halves)]
    outpart, denpart = kern_c(logits, srcw, dstw, mpart, *xl_halves)
    return outpart, denpart


def kernel(x, adj, W1l, W1r, att1, b1, bn_g, bn_b, W2l, W2r, att2, b2,
           W3l, W3r, att3, b3):
    loop = jnp.arange(_N, dtype=adj.dtype)
    pad = jnp.zeros((_NW * _EPW - _ETOT,), adj.dtype)
    srcw = jnp.concatenate([adj[0], loop, pad]).reshape(_NW, _NB, _B)
    dstw = jnp.concatenate([adj[1], loop, pad]).reshape(_NW, _NB, _B)

    xl1, xr1 = pl.pallas_call(
        _tc0_body,
        out_shape=[_f32(_N, 256), _f32(_N, 256)],
    )(x, W1l, W1r)

    op1, den1 = _gat_layer(xl1, xr1, att1, srcw, dstw, _kernel_a256,
                           _kernel_c2, 2)

    xl2, xr2, xl3, xr3 = pl.pallas_call(
        _tc1_body,
        out_shape=[_f32(_N, 128)] * 4,
    )(op1, den1, b1, bn_g, bn_b, W2l, W2r, W3l, W3r)

    op2, den2 = _gat_layer(xl2, xr2, att2, srcw, dstw, _kernel_a128,
                           _kernel_c1, 1)
    op3, den3 = _gat_layer(xl3, xr3, att3, srcw, dstw, _kernel_a128,
                           _kernel_c1, 1)

    mu, logvar = pl.pallas_call(
        _tc2_body,
        out_shape=[_f32(_N, 128), _f32(_N, 128)],
    )(op2, den2, op3, den3, b2, b3)
    return (mu, logvar)


# speedup vs baseline: 7.0966x; 1.0007x over previous
import functools
import jax
import jax.numpy as jnp
from jax import lax
from jax.experimental import pallas as pl
from jax.experimental.pallas import tpu as pltpu
from jax.experimental.pallas import tpu_sc as plsc

_N = 10000
_E = 320000
_ETOT = _E + _N
_NW = 32
_B = 64
_NB = -(-_ETOT // (_NW * _B))
_EPW = _NB * _B
_NPAD = 10240
_SL = _NPAD // 16

_mesh = plsc.VectorSubcoreMesh(core_axis_name="c", subcore_axis_name="s",
                               num_cores=2, num_subcores=16)
_params = pltpu.CompilerParams(needs_layout_passes=False)


def _f32(*shape):
    return jax.ShapeDtypeStruct(shape, jnp.float32)


def _make_kernel_a(H):
    HC = H // 16

    @functools.partial(
        pl.kernel,
        mesh=_mesh,
        compiler_params=_params,
        out_type=[_f32(_NW, _EPW), _f32(2, _NPAD)],
        scratch_types=[
            pltpu.VMEM((_B,), jnp.int32),
            pltpu.VMEM((_B,), jnp.int32),
            pltpu.VMEM((_B,), jnp.int32),
            pltpu.VMEM((_B,), jnp.int32),
            pltpu.VMEM((_B, H), jnp.float32),
            pltpu.VMEM((_B, H), jnp.float32),
            pltpu.VMEM((_B, H), jnp.float32),
            pltpu.VMEM((_B, H), jnp.float32),
            pltpu.VMEM((H,), jnp.float32),
            pltpu.VMEM((_EPW,), jnp.float32),
            pltpu.VMEM((_NPAD,), jnp.float32),
            pltpu.VMEM((256,), jnp.float32),
            pltpu.VMEM((_SL,), jnp.float32),
            pltpu.VMEM_SHARED((16, _NPAD), jnp.float32),
            pltpu.SemaphoreType.DMA,
            pltpu.SemaphoreType.DMA,
            pltpu.SemaphoreType.DMA,
            pltpu.SemaphoreType.DMA,
        ],
    )
    def kern(xl, xr, att, srcw, dstw, logits_o, mpart_o,
             idx_s0, idx_s1, idx_d0, idx_d1, rows_l0, rows_l1,
             rows_r0, rows_r1, att_v, logit_loc, m_loc,
             tbuf, red_out, stage, sl0, sl1, sr0, sr1):
        c = lax.axis_index("c")
        s = lax.axis_index("s")
        wid = s * 2 + c
        slots = ((idx_s0, idx_d0, rows_l0, rows_r0, sl0, sr0),
                 (idx_s1, idx_d1, rows_l1, rows_r1, sl1, sr1))

        pltpu.sync_copy(att, att_v)

        neg = jnp.full((16,), -1e30, jnp.float32)

        def init_m(i, _):
            m_loc[pl.ds(i * 16, 16)] = neg
            return 0
        lax.fori_loop(0, _NPAD // 16, init_m, 0)

        att_ch = [att_v[pl.ds(16 * k, 16)] for k in range(HC)]
        iot = lax.iota(jnp.int32, 16)

        def fire(j, slot):
            idx_s, idx_d, rows_l, rows_r, sem_l, sem_r = slot
            pltpu.sync_copy(srcw.at[wid, j], idx_s)
            pltpu.sync_copy(dstw.at[wid, j], idx_d)
            pltpu.async_copy(xl.at[idx_s], rows_l, sem_l)
            pltpu.async_copy(xr.at[idx_d], rows_r, sem_r)

        def process(j, slot):
            idx_s, idx_d, rows_l, rows_r, sem_l, sem_r = slot
            pltpu.make_async_copy(xl.at[idx_s], rows_l, sem_l).wait()
            pltpu.make_async_copy(xr.at[idx_d], rows_r, sem_r).wait()

            def group_body(g, _):
                for e in range(16):
                    acc = jnp.zeros((16,), jnp.float32)
                    for k in range(HC):
                        a = (rows_l[g * 16 + e, pl.ds(16 * k, 16)]
                             + rows_r[g * 16 + e, pl.ds(16 * k, 16)])
                        a = jnp.maximum(a, 0.2 * a)
                        acc = acc + a * att_ch[k]
                    tbuf[pl.ds(e * 16, 16)] = acc
                lg = jnp.zeros((16,), jnp.float32)
                for k in range(16):
                    lg = lg + plsc.load_gather(tbuf, [iot * 16 + k])
                ge = wid * _EPW + j * _B + g * 16 + iot
                lg = jnp.where(ge < _ETOT, lg,
                               jnp.full((16,), -1e30, jnp.float32))
                logit_loc[pl.ds(j * _B + g * 16, 16)] = lg
                dvec = idx_d[pl.ds(g * 16, 16)]
                cnt, _ = plsc.scan_count(dvec)
                nodup = jnp.max(cnt) < 2

                @pl.when(nodup)
                def _():
                    mv = plsc.load_gather(m_loc, [dvec])
                    plsc.store_scatter(m_loc, [dvec], jnp.maximum(mv, lg))

                @pl.when(jnp.logical_not(nodup))
                def _():
                    for k in range(16):
                        mv = plsc.load_gather(m_loc, [dvec])
                        plsc.store_scatter(m_loc, [dvec],
                                           jnp.maximum(mv, lg),
                                           mask=iot == k)
                return 0
            lax.fori_loop(0, _B // 16, group_body, 0)

        fire(0, slots[0])
        fire(1, slots[1])

        def pipe_body(i, _):
            for p in range(2):
                j = 2 * i + p
                process(j, slots[p])

                @pl.when(j + 2 < _NB)
                def _():
                    fire(j + 2, slots[p])
            return 0
        lax.fori_loop(0, _NB // 2, pipe_body, 0)

        pltpu.sync_copy(logit_loc, logits_o.at[wid])

        pltpu.sync_copy(m_loc, stage.at[s])
        plsc.subcore_barrier()
        for r in range(16):
            pltpu.sync_copy(stage.at[r, pl.ds(s * _SL, _SL)],
                            m_loc.at[pl.ds(r * _SL, _SL)])

        def red_body(i, _):
            acc = m_loc[pl.ds(16 * i, 16)]
            for r in range(1, 16):
                acc = jnp.maximum(acc, m_loc[pl.ds(r * _SL + 16 * i, 16)])
            red_out[pl.ds(16 * i, 16)] = acc
            return 0
        lax.fori_loop(0, _SL // 16, red_body, 0)
        pltpu.sync_copy(red_out, mpart_o.at[c, pl.ds(s * _SL, _SL)])

    return kern


def _make_kernel_c(halves):
    @functools.partial(
        pl.kernel,
        mesh=_mesh,
        compiler_params=_params,
        out_type=[_f32(2, halves, _NPAD, 128), _f32(2, _NPAD)],
        scratch_types=[
            pltpu.VMEM((_B,), jnp.int32),
            pltpu.VMEM((_B,), jnp.int32),
            pltpu.VMEM((_B,), jnp.int32),
            pltpu.VMEM((_B,), jnp.int32),
            pltpu.VMEM((_B,), jnp.float32),
            pltpu.VMEM((_B,), jnp.float32),
            pltpu.VMEM((_NPAD,), jnp.float32),
            pltpu.VMEM((_NPAD,), jnp.float32),
            pltpu.VMEM((_B, 128), jnp.float32),
            pltpu.VMEM((_B, 128), jnp.float32),
            pltpu.VMEM_SHARED((_NPAD, 128), jnp.float32),
            pltpu.VMEM_SHARED((_NPAD,), jnp.float32),
            pltpu.SemaphoreType.DMA,
            pltpu.SemaphoreType.DMA,
            pltpu.SemaphoreType.DMA,
            pltpu.SemaphoreType.DMA,
        ],
    )
    def kern(*args):
        (lg_i, srcw, dstw, mpart) = args[:4]
        xl_halves = args[4:4 + halves]
        outpart_o = args[4 + halves]
        denpart_o = args[5 + halves]
        (idx_s0, idx_s1, idx_d0, idx_d1, ex0, ex1, mf, m2,
         rows0, rows1, out_sh, den_sh, sem0, sem1, sc0, sc1) = \
            args[6 + halves:]

        c = lax.axis_index("c")
        s = lax.axis_index("s")
        wid = s * 2 + c
        slots = ((idx_s0, idx_d0, ex0, rows0, sem0, sc0),
                 (idx_s1, idx_d1, ex1, rows1, sem1, sc1))

        pltpu.sync_copy(mpart.at[0], mf)
        pltpu.sync_copy(mpart.at[1], m2)

        zero = jnp.zeros((16,), jnp.float32)

        def mmax(i, _):
            mf[pl.ds(16 * i, 16)] = jnp.maximum(mf[pl.ds(16 * i, 16)],
                                                m2[pl.ds(16 * i, 16)])
            return 0
        lax.fori_loop(0, _NPAD // 16, mmax, 0)

        for half in range(halves):
            xlh = xl_halves[half]

            def wait_scatter(slot):
                idx_s, idx_d, exb, rows, sem, sem_sc = slot
                pltpu.make_async_copy(rows, out_sh.at[idx_d], sem_sc).wait()
                if half == 0:
                    pltpu.make_async_copy(exb, den_sh.at[idx_d],
                                          sem_sc).wait()

            def fire(j, slot):
                idx_s, idx_d, exb, rows, sem, sem_sc = slot
                pltpu.sync_copy(srcw.at[wid, j], idx_s)
                pltpu.sync_copy(dstw.at[wid, j], idx_d)
                pltpu.sync_copy(lg_i.at[wid, pl.ds(j * _B, _B)], exb)
                pltpu.async_copy(xlh.at[idx_s], rows, sem)

            def process(j, slot):
                idx_s, idx_d, exb, rows, sem, sem_sc = slot
                pltpu.make_async_copy(xlh.at[idx_s], rows, sem).wait()

                def group_body(g, _):
                    dvec = idx_d[pl.ds(g * 16, 16)]
                    lgv = exb[pl.ds(g * 16, 16)]
                    mg = plsc.load_gather(mf, [dvec])
                    alv = jnp.exp(lgv - mg)
                    if half == 0:
                        exb[pl.ds(g * 16, 16)] = alv
                    for e in range(16):
                        a = alv[e]
                        for k in range(8):
                            rows[g * 16 + e, pl.ds(16 * k, 16)] = (
                                rows[g * 16 + e, pl.ds(16 * k, 16)] * a)
                    return 0
                lax.fori_loop(0, _B // 16, group_body, 0)
                pltpu.async_copy(rows, out_sh.at[idx_d], sem_sc, add=True)
                if half == 0:
                    pltpu.async_copy(exb, den_sh.at[idx_d], sem_sc,
                                     add=True)

            for e in range(16):
                for k in range(8):
                    rows0[e, pl.ds(16 * k, 16)] = zero

            def zrow(i, _):
                pltpu.sync_copy(rows0.at[pl.ds(0, 16), :],
                                out_sh.at[pl.ds(s * _SL + i * 16, 16), :])
                return 0
            lax.fori_loop(0, _SL // 16, zrow, 0)
            if half == 0:
                def zden(i, _):
                    pltpu.sync_copy(rows0.at[0, pl.ds(0, 16)],
                                    den_sh.at[pl.ds(s * _SL + i * 16, 16)])
                    return 0
                lax.fori_loop(0, _SL // 16, zden, 0)
            plsc.subcore_barrier()

            fire(0, slots[0])
            fire(1, slots[1])

            def pipe_body(i, _):
                for p in range(2):
                    j = 2 * i + p
                    process(j, slots[p])

                    @pl.when(j + 2 < _NB)
                    def _():
                        wait_scatter(slots[p])
                        fire(j + 2, slots[p])
                return 0
            lax.fori_loop(0, _NB // 2, pipe_body, 0)
            wait_scatter(slots[0])
            wait_scatter(slots[1])
            plsc.subcore_barrier()
            pltpu.sync_copy(out_sh.at[pl.ds(s * _SL, _SL), :],
                            outpart_o.at[c, half, pl.ds(s * _SL, _SL), :])
            if half == 0:
                pltpu.sync_copy(den_sh.at[pl.ds(s * _SL, _SL)],
                                denpart_o.at[c, pl.ds(s * _SL, _SL)])
            plsc.subcore_barrier()

    return kern


_kernel_a256 = _make_kernel_a(256)
_kernel_a128 = _make_kernel_a(128)
_kernel_c2 = _make_kernel_c(2)
_kernel_c1 = _make_kernel_c(1)


def _tc0_body(x_ref, wl_ref, wr_ref, xl_ref, xr_ref):
    xl_ref[...] = jnp.dot(x_ref[...], wl_ref[...],
                          preferred_element_type=jnp.float32)
    xr_ref[...] = jnp.dot(x_ref[...], wr_ref[...],
                          preferred_element_type=jnp.float32)


def _tc1_body(op_ref, den_ref, b1_ref, g_ref, bb_ref, w2l_ref, w2r_ref,
              w3l_ref, w3r_ref, xl2_ref, xr2_ref, xl3_ref, xr3_ref):
    den = (den_ref[0, :_N] + den_ref[1, :_N] + 1e-16)[:, None]
    h = jnp.concatenate(
        [op_ref[0, 0, :_N, :] + op_ref[1, 0, :_N, :],
         op_ref[0, 1, :_N, :] + op_ref[1, 1, :_N, :]], axis=1)
    h = h / den + b1_ref[...]
    mean = jnp.mean(h, axis=0, keepdims=True)
    var = jnp.mean((h - mean) ** 2, axis=0, keepdims=True)
    h = g_ref[...] * (h - mean) / jnp.sqrt(var + 1e-5) + bb_ref[...]
    h = jnp.maximum(h, 0.0)
    xl2_ref[...] = jnp.dot(h, w2l_ref[...], preferred_element_type=jnp.float32)
    xr2_ref[...] = jnp.dot(h, w2r_ref[...], preferred_element_type=jnp.float32)
    xl3_ref[...] = jnp.dot(h, w3l_ref[...], preferred_element_type=jnp.float32)
    xr3_ref[...] = jnp.dot(h, w3r_ref[...], preferred_element_type=jnp.float32)


def _tc2_body(p2_ref, d2_ref, p3_ref, d3_ref, b2_ref, b3_ref, mu_ref,
              lv_ref):
    den2 = (d2_ref[0, :_N] + d2_ref[1, :_N] + 1e-16)[:, None]
    den3 = (d3_ref[0, :_N] + d3_ref[1, :_N] + 1e-16)[:, None]
    mu_ref[...] = ((p2_ref[0, 0, :_N, :] + p2_ref[1, 0, :_N, :]) / den2
                   + b2_ref[...])
    lv_ref[...] = ((p3_ref[0, 0, :_N, :] + p3_ref[1, 0, :_N, :]) / den3
                   + b3_ref[...])


def _gat_layer(xl, xr, att, srcw, dstw, kern_a, kern_c, halves):
    logits, mpart = kern_a(xl, xr, att, srcw, dstw)
    xl_halves = [xl[:, 128 * h:128 * (h + 1)] for h in range(halves)]
    outpart, denpart = kern_c(logits, srcw, dstw, mpart, *xl_halves)
    return outpart, denpart


def kernel(x, adj, W1l, W1r, att1, b1, bn_g, bn_b, W2l, W2r, att2, b2,
           W3l, W3r, att3, b3):
    loop = jnp.arange(_N, dtype=adj.dtype)
    pad = jnp.zeros((_NW * _EPW - _ETOT,), adj.dtype)
    srcw = jnp.concatenate([adj[0], loop, pad]).reshape(_NW, _NB, _B)
    dstw = jnp.concatenate([adj[1], loop, pad]).reshape(_NW, _NB, _B)

    xl1, xr1 = pl.pallas_call(
        _tc0_body,
        out_shape=[_f32(_N, 256), _f32(_N, 256)],
    )(x, W1l, W1r)

    op1, den1 = _gat_layer(xl1, xr1, att1, srcw, dstw, _kernel_a256,
                           _kernel_c2, 2)

    xl2, xr2, xl3, xr3 = pl.pallas_call(
        _tc1_body,
        out_shape=[_f32(_N, 128)] * 4,
    )(op1, den1, b1, bn_g, bn_b, W2l, W2r, W3l, W3r)

    op2, den2 = _gat_layer(xl2, xr2, att2, srcw, dstw, _kernel_a128,
                           _kernel_c1, 1)
    op3, den3 = _gat_layer(xl3, xr3, att3, srcw, dstw, _kernel_a128,
                           _kernel_c1, 1)

    mu, logvar = pl.pallas_call(
        _tc2_body,
        out_shape=[_f32(_N, 128), _f32(_N, 128)],
    )(op2, den2, op3, den3, b2, b3)
    return (mu, logvar)


# speedup vs baseline: 8.0998x; 1.1414x over previous
import functools
import jax
import jax.numpy as jnp
from jax import lax
from jax.experimental import pallas as pl
from jax.experimental.pallas import tpu as pltpu
from jax.experimental.pallas import tpu_sc as plsc

_N = 10000
_E = 320000
_ETOT = _E + _N
_NW = 32
_B = 64
_NB = -(-_ETOT // (_NW * _B))
_EPW = _NB * _B
_NPAD = 10240
_SL = _NPAD // 16

_mesh = plsc.VectorSubcoreMesh(core_axis_name="c", subcore_axis_name="s",
                               num_cores=2, num_subcores=16)
_params = pltpu.CompilerParams(needs_layout_passes=False)


def _f32(*shape):
    return jax.ShapeDtypeStruct(shape, jnp.float32)


def _make_kernel_a(H):
    HC = H // 16

    @functools.partial(
        pl.kernel,
        mesh=_mesh,
        compiler_params=_params,
        out_type=[_f32(_NW, _EPW), _f32(2, _NPAD)],
        scratch_types=[
            pltpu.VMEM((_B,), jnp.int32),
            pltpu.VMEM((_B,), jnp.int32),
            pltpu.VMEM((_NB, _B), jnp.int32),
            pltpu.VMEM((_B, H), jnp.float32),
            pltpu.VMEM((_B, H), jnp.float32),
            pltpu.VMEM((_B, H), jnp.float32),
            pltpu.VMEM((_B, H), jnp.float32),
            pltpu.VMEM((H,), jnp.float32),
            pltpu.VMEM((_EPW,), jnp.float32),
            pltpu.VMEM((_NPAD,), jnp.float32),
            pltpu.VMEM((256,), jnp.float32),
            pltpu.VMEM((_SL,), jnp.float32),
            pltpu.VMEM_SHARED((16, _NPAD), jnp.float32),
            pltpu.SemaphoreType.DMA,
            pltpu.SemaphoreType.DMA,
            pltpu.SemaphoreType.DMA,
            pltpu.SemaphoreType.DMA,
        ],
    )
    def kern(xl, xr, att, srcw, dstw, logits_o, mpart_o,
             idx_s0, idx_s1, dst_loc, rows_l0, rows_l1,
             rows_r0, rows_r1, att_v, logit_loc, m_loc,
             tbuf, red_out, stage, sl0, sl1, sr0, sr1):
        c = lax.axis_index("c")
        s = lax.axis_index("s")
        wid = s * 2 + c
        slots = ((idx_s0, rows_l0, rows_r0, sl0, sr0),
                 (idx_s1, rows_l1, rows_r1, sl1, sr1))

        pltpu.sync_copy(dstw.at[wid], dst_loc)
        pltpu.sync_copy(att, att_v)

        neg = jnp.full((16,), -1e30, jnp.float32)

        def init_m(i, _):
            m_loc[pl.ds(i * 16, 16)] = neg
            return 0
        lax.fori_loop(0, _NPAD // 16, init_m, 0)

        att_ch = [att_v[pl.ds(16 * k, 16)] for k in range(HC)]
        iot = lax.iota(jnp.int32, 16)

        def fire(j, slot):
            idx_s, rows_l, rows_r, sem_l, sem_r = slot
            pltpu.sync_copy(srcw.at[wid, j], idx_s)
            pltpu.async_copy(xl.at[idx_s], rows_l, sem_l)
            pltpu.async_copy(xr.at[dst_loc.at[j]], rows_r, sem_r)

        def process(j, slot):
            idx_s, rows_l, rows_r, sem_l, sem_r = slot
            pltpu.make_async_copy(xl.at[idx_s], rows_l, sem_l).wait()
            pltpu.make_async_copy(xr.at[dst_loc.at[j]], rows_r, sem_r).wait()

            def group_body(g, _):
                for e in range(16):
                    acc = jnp.zeros((16,), jnp.float32)
                    for k in range(HC):
                        a = (rows_l[g * 16 + e, pl.ds(16 * k, 16)]
                             + rows_r[g * 16 + e, pl.ds(16 * k, 16)])
                        a = jnp.maximum(a, 0.2 * a)
                        acc = acc + a * att_ch[k]
                    tbuf[pl.ds(e * 16, 16)] = acc
                lg = jnp.zeros((16,), jnp.float32)
                for k in range(16):
                    lg = lg + plsc.load_gather(tbuf, [iot * 16 + k])
                ge = wid * _EPW + j * _B + g * 16 + iot
                lg = jnp.where(ge < _ETOT, lg,
                               jnp.full((16,), -1e30, jnp.float32))
                logit_loc[pl.ds(j * _B + g * 16, 16)] = lg
                dvec = dst_loc[j, pl.ds(g * 16, 16)]
                cnt, _ = plsc.scan_count(dvec)
                nodup = jnp.max(cnt) < 2

                @pl.when(nodup)
                def _():
                    mv = plsc.load_gather(m_loc, [dvec])
                    plsc.store_scatter(m_loc, [dvec], jnp.maximum(mv, lg))

                @pl.when(jnp.logical_not(nodup))
                def _():
                    for k in range(16):
                        mv = plsc.load_gather(m_loc, [dvec])
                        plsc.store_scatter(m_loc, [dvec],
                                           jnp.maximum(mv, lg),
                                           mask=iot == k)
                return 0
            lax.fori_loop(0, _B // 16, group_body, 0)

        fire(0, slots[0])
        fire(1, slots[1])

        def pipe_body(i, _):
            for p in range(2):
                j = 2 * i + p
                process(j, slots[p])

                @pl.when(j + 2 < _NB)
                def _():
                    fire(j + 2, slots[p])
            return 0
        lax.fori_loop(0, _NB // 2, pipe_body, 0)

        pltpu.sync_copy(logit_loc, logits_o.at[wid])

        pltpu.sync_copy(m_loc, stage.at[s])
        plsc.subcore_barrier()
        for r in range(16):
            pltpu.sync_copy(stage.at[r, pl.ds(s * _SL, _SL)],
                            m_loc.at[pl.ds(r * _SL, _SL)])

        def red_body(i, _):
            acc = m_loc[pl.ds(16 * i, 16)]
            for r in range(1, 16):
                acc = jnp.maximum(acc, m_loc[pl.ds(r * _SL + 16 * i, 16)])
            red_out[pl.ds(16 * i, 16)] = acc
            return 0
        lax.fori_loop(0, _SL // 16, red_body, 0)
        pltpu.sync_copy(red_out, mpart_o.at[c, pl.ds(s * _SL, _SL)])

    return kern


def _make_kernel_c(halves):
    @functools.partial(
        pl.kernel,
        mesh=_mesh,
        compiler_params=_params,
        out_type=[_f32(2, halves, _NPAD, 128), _f32(2, _NPAD)],
        scratch_types=[
            pltpu.VMEM((_B,), jnp.int32),
            pltpu.VMEM((_B,), jnp.int32),
            pltpu.VMEM((_B,), jnp.int32),
            pltpu.VMEM((_B,), jnp.int32),
            pltpu.VMEM((_B,), jnp.float32),
            pltpu.VMEM((_B,), jnp.float32),
            pltpu.VMEM((_EPW,), jnp.float32),
            pltpu.VMEM((_NPAD,), jnp.float32),
            pltpu.VMEM((_NPAD,), jnp.float32),
            pltpu.VMEM((_B, 128), jnp.float32),
            pltpu.VMEM((_B, 128), jnp.float32),
            pltpu.VMEM_SHARED((_NPAD, 128), jnp.float32),
            pltpu.VMEM_SHARED((_NPAD,), jnp.float32),
            pltpu.SemaphoreType.DMA,
            pltpu.SemaphoreType.DMA,
            pltpu.SemaphoreType.DMA,
            pltpu.SemaphoreType.DMA,
        ],
    )
    def kern(*args):
        (lg_i, srcw, dstw, mpart) = args[:4]
        xl_halves = args[4:4 + halves]
        outpart_o = args[4 + halves]
        denpart_o = args[5 + halves]
        (idx_s0, idx_s1, idx_d0, idx_d1, ex0, ex1, lg_loc, mf, m2,
         rows0, rows1, out_sh, den_sh, sem0, sem1, sc0, sc1) = \
            args[6 + halves:]

        c = lax.axis_index("c")
        s = lax.axis_index("s")
        wid = s * 2 + c
        slots = ((idx_s0, idx_d0, ex0, rows0, sem0, sc0),
                 (idx_s1, idx_d1, ex1, rows1, sem1, sc1))

        pltpu.sync_copy(lg_i.at[wid], lg_loc)
        pltpu.sync_copy(mpart.at[0], mf)
        pltpu.sync_copy(mpart.at[1], m2)

        zero = jnp.zeros((16,), jnp.float32)

        def mmax(i, _):
            mf[pl.ds(16 * i, 16)] = jnp.maximum(mf[pl.ds(16 * i, 16)],
                                                m2[pl.ds(16 * i, 16)])
            return 0
        lax.fori_loop(0, _NPAD // 16, mmax, 0)

        for half in range(halves):
            xlh = xl_halves[half]

            def wait_scatter(slot):
                idx_s, idx_d, exb, rows, sem, sem_sc = slot
                pltpu.make_async_copy(rows, out_sh.at[idx_d], sem_sc).wait()
                if half == 0:
                    pltpu.make_async_copy(exb, den_sh.at[idx_d],
                                          sem_sc).wait()

            def fire(j, slot):
                idx_s, idx_d, exb, rows, sem, sem_sc = slot
                pltpu.sync_copy(srcw.at[wid, j], idx_s)
                pltpu.sync_copy(dstw.at[wid, j], idx_d)
                pltpu.async_copy(xlh.at[idx_s], rows, sem)

            def process(j, slot):
                idx_s, idx_d, exb, rows, sem, sem_sc = slot
                pltpu.make_async_copy(xlh.at[idx_s], rows, sem).wait()

                def group_body(g, _):
                    dvec = idx_d[pl.ds(g * 16, 16)]
                    lgv = lg_loc[pl.ds(j * _B + g * 16, 16)]
                    mg = plsc.load_gather(mf, [dvec])
                    alv = jnp.exp(lgv - mg)
                    if half == 0:
                        exb[pl.ds(g * 16, 16)] = alv
                    for e in range(16):
                        a = alv[e]
                        for k in range(8):
                            rows[g * 16 + e, pl.ds(16 * k, 16)] = (
                                rows[g * 16 + e, pl.ds(16 * k, 16)] * a)
                    return 0
                lax.fori_loop(0, _B // 16, group_body, 0)
                pltpu.async_copy(rows, out_sh.at[idx_d], sem_sc, add=True)
                if half == 0:
                    pltpu.async_copy(exb, den_sh.at[idx_d], sem_sc,
                                     add=True)

            for e in range(16):
                for k in range(8):
                    rows0[e, pl.ds(16 * k, 16)] = zero

            def zrow(i, _):
                pltpu.sync_copy(rows0.at[pl.ds(0, 16), :],
                                out_sh.at[pl.ds(s * _SL + i * 16, 16), :])
                return 0
            lax.fori_loop(0, _SL // 16, zrow, 0)
            if half == 0:
                def zden(i, _):
                    pltpu.sync_copy(rows0.at[0, pl.ds(0, 16)],
                                    den_sh.at[pl.ds(s * _SL + i * 16, 16)])
                    return 0
                lax.fori_loop(0, _SL // 16, zden, 0)
            plsc.subcore_barrier()

            fire(0, slots[0])
            fire(1, slots[1])

            def pipe_body(i, _):
                for p in range(2):
                    j = 2 * i + p
                    process(j, slots[p])

                    @pl.when(j + 2 < _NB)
                    def _():
                        wait_scatter(slots[p])
                        fire(j + 2, slots[p])
                return 0
            lax.fori_loop(0, _NB // 2, pipe_body, 0)
            wait_scatter(slots[0])
            wait_scatter(slots[1])
            plsc.subcore_barrier()
            pltpu.sync_copy(out_sh.at[pl.ds(s * _SL, _SL), :],
                            outpart_o.at[c, half, pl.ds(s * _SL, _SL), :])
            if half == 0:
                pltpu.sync_copy(den_sh.at[pl.ds(s * _SL, _SL)],
                                denpart_o.at[c, pl.ds(s * _SL, _SL)])
            plsc.subcore_barrier()

    return kern


_kernel_a256 = _make_kernel_a(256)
_kernel_a128 = _make_kernel_a(128)
_kernel_c2 = _make_kernel_c(2)
_kernel_c1 = _make_kernel_c(1)


def _tc0_body(x_ref, wl_ref, wr_ref, xl_ref, xr_ref):
    xl_ref[...] = jnp.dot(x_ref[...], wl_ref[...],
                          preferred_element_type=jnp.float32)
    xr_ref[...] = jnp.dot(x_ref[...], wr_ref[...],
                          preferred_element_type=jnp.float32)


def _tc1_body(op_ref, den_ref, b1_ref, g_ref, bb_ref, w2l_ref, w2r_ref,
              w3l_ref, w3r_ref, xl2_ref, xr2_ref, xl3_ref, xr3_ref):
    den = (den_ref[0, :_N] + den_ref[1, :_N] + 1e-16)[:, None]
    h = jnp.concatenate(
        [op_ref[0, 0, :_N, :] + op_ref[1, 0, :_N, :],
         op_ref[0, 1, :_N, :] + op_ref[1, 1, :_N, :]], axis=1)
    h = h / den + b1_ref[...]
    mean = jnp.mean(h, axis=0, keepdims=True)
    var = jnp.mean((h - mean) ** 2, axis=0, keepdims=True)
    h = g_ref[...] * (h - mean) / jnp.sqrt(var + 1e-5) + bb_ref[...]
    h = jnp.maximum(h, 0.0)
    xl2_ref[...] = jnp.dot(h, w2l_ref[...], preferred_element_type=jnp.float32)
    xr2_ref[...] = jnp.dot(h, w2r_ref[...], preferred_element_type=jnp.float32)
    xl3_ref[...] = jnp.dot(h, w3l_ref[...], preferred_element_type=jnp.float32)
    xr3_ref[...] = jnp.dot(h, w3r_ref[...], preferred_element_type=jnp.float32)


def _tc2_body(p2_ref, d2_ref, p3_ref, d3_ref, b2_ref, b3_ref, mu_ref,
              lv_ref):
    den2 = (d2_ref[0, :_N] + d2_ref[1, :_N] + 1e-16)[:, None]
    den3 = (d3_ref[0, :_N] + d3_ref[1, :_N] + 1e-16)[:, None]
    mu_ref[...] = ((p2_ref[0, 0, :_N, :] + p2_ref[1, 0, :_N, :]) / den2
                   + b2_ref[...])
    lv_ref[...] = ((p3_ref[0, 0, :_N, :] + p3_ref[1, 0, :_N, :]) / den3
                   + b3_ref[...])


def _gat_layer(xl, xr, att, srcw, dstw, kern_a, kern_c, halves):
    logits, mpart = kern_a(xl, xr, att, srcw, dstw)
    xl_halves = [xl[:, 128 * h:128 * (h + 1)] for h in range(halves)]
    outpart, denpart = kern_c(logits, srcw, dstw, mpart, *xl_halves)
    return outpart, denpart


def kernel(x, adj, W1l, W1r, att1, b1, bn_g, bn_b, W2l, W2r, att2, b2,
           W3l, W3r, att3, b3):
    loop = jnp.arange(_N, dtype=adj.dtype)
    pad = jnp.zeros((_NW * _EPW - _ETOT,), adj.dtype)
    srcw = jnp.concatenate([adj[0], loop, pad]).reshape(_NW, _NB, _B)
    dstw = jnp.concatenate([adj[1], loop, pad]).reshape(_NW, _NB, _B)

    xl1, xr1 = pl.pallas_call(
        _tc0_body,
        out_shape=[_f32(_N, 256), _f32(_N, 256)],
    )(x, W1l, W1r)

    op1, den1 = _gat_layer(xl1, xr1, att1, srcw, dstw, _kernel_a256,
                           _kernel_c2, 2)

    xl2, xr2, xl3, xr3 = pl.pallas_call(
        _tc1_body,
        out_shape=[_f32(_N, 128)] * 4,
    )(op1, den1, b1, bn_g, bn_b, W2l, W2r, W3l, W3r)

    op2, den2 = _gat_layer(xl2, xr2, att2, srcw, dstw, _kernel_a128,
                           _kernel_c1, 1)
    op3, den3 = _gat_layer(xl3, xr3, att3, srcw, dstw, _kernel_a128,
                           _kernel_c1, 1)

    mu, logvar = pl.pallas_call(
        _tc2_body,
        out_shape=[_f32(_N, 128), _f32(_N, 128)],
    )(op2, den2, op3, den3, b2, b3)
    return (mu, logvar)
